# baseline (device time: 73892 ns/iter reference)
import functools
import os

import jax
import jax.numpy as jnp
from jax import lax
from jax.experimental import pallas as pl
from jax.experimental.pallas import tpu as pltpu

N_DEV = 4
_NO_COMM = os.environ.get("KERNEL_NO_COMM") == "1"
B, SQ, SKV_LOC, HQ, DH = 2, 512, 512, 8, 64
D_MODEL = 768
BLK = 64


def kernel(x, Wq, K_ext, V_ext, Wo):
    xb = x.astype(jnp.bfloat16)
    wqb = Wq.astype(jnp.bfloat16)
    kb = K_ext.astype(jnp.bfloat16).reshape(B, SKV_LOC, HQ * DH).transpose(0, 2, 1)
    vb = V_ext.astype(jnp.bfloat16).reshape(B, SKV_LOC, HQ * DH)
    wob = Wo.astype(jnp.bfloat16)

    def body(x_ref, wq_ref, k_ref, v_ref, wo_ref, out_ref,
             kb_buf, vb_buf, send_sems, recv_sems):
        my = lax.axis_index("i")
        left = (my - 1) % N_DEV
        right = (my + 1) % N_DEV

        if not _NO_COMM:
            barrier_sem = pltpu.get_barrier_semaphore()
            for nbr in (left, right):
                pl.semaphore_signal(
                    barrier_sem, inc=1,
                    device_id=(nbr,), device_id_type=pl.DeviceIdType.MESH,
                )
            pl.semaphore_wait(barrier_sem, 2)

        def copy_to(target, sem_base):
            ops = []
            for buf, slot in ((kb_buf, 0), (vb_buf, 1)):
                rdma = pltpu.make_async_remote_copy(
                    src_ref=buf,
                    dst_ref=buf,
                    send_sem=send_sems.at[sem_base + slot],
                    recv_sem=recv_sems.at[slot],
                    device_id=(target,),
                    device_id_type=pl.DeviceIdType.MESH,
                )
                ops.append(rdma)
            return ops

        def recv_descs():
            return copy_to(left, 0)

        if _NO_COMM:
            kb_buf[...] = k_ref[...]
            vb_buf[...] = v_ref[...]
        else:
            @pl.when(my == 0)
            def _():
                kb_buf[...] = k_ref[...]
                vb_buf[...] = v_ref[...]
                sends = copy_to(1, 0) + copy_to(3, 2)
                for op in sends:
                    op.start()
                for op in sends:
                    op.wait_send()

            @pl.when(my == 1)
            def _():
                for op in recv_descs():
                    op.wait_recv()
                fwd = copy_to(2, 0)
                for op in fwd:
                    op.start()
                for op in fwd:
                    op.wait_send()

            @pl.when((my == 2) | (my == 3))
            def _():
                for op in recv_descs():
                    op.wait_recv()

        rowb = lax.broadcasted_iota(jnp.int32, (SQ, SKV_LOC), 0) // BLK
        colb = lax.broadcasted_iota(jnp.int32, (SQ, SKV_LOC), 1) // BLK
        mask = colb <= rowb

        for b in range(B):
            x_b = x_ref[b]
            q_full = jnp.dot(x_b, wq_ref[...],
                             preferred_element_type=jnp.float32)
            acc = jnp.zeros((SQ, D_MODEL), jnp.float32)
            for h in range(HQ):
                q_h = q_full[:, h * DH:(h + 1) * DH].astype(jnp.bfloat16)
                kt_h = kb_buf[b, h * DH:(h + 1) * DH, :]
                s = jnp.dot(q_h, kt_h,
                            preferred_element_type=jnp.float32) * 0.125
                s = jnp.where(mask, s, -1e9)
                m = jnp.max(s, axis=-1, keepdims=True)
                w = jnp.exp(s - m)
                w = w / jnp.sum(w, axis=-1, keepdims=True)
                v_h = vb_buf[b, :, h * DH:(h + 1) * DH]
                ctx = jnp.dot(w.astype(jnp.bfloat16), v_h,
                              preferred_element_type=jnp.float32)
                acc = acc + jnp.dot(
                    ctx.astype(jnp.bfloat16), wo_ref[h * DH:(h + 1) * DH, :],
                    preferred_element_type=jnp.float32)
            out_ref[b] = acc

        if not _NO_COMM:
            @functools.partial(pl.run_scoped,
                               second_barrier=pltpu.SemaphoreType.REGULAR)
            def _(second_barrier):
                for nbr in (left, right):
                    pl.semaphore_signal(
                        second_barrier, inc=1,
                        device_id=(nbr,), device_id_type=pl.DeviceIdType.MESH,
                    )
                pl.semaphore_wait(second_barrier, 2)

    return pl.pallas_call(
        body,
        out_shape=jax.ShapeDtypeStruct((B, SQ, D_MODEL), jnp.float32),
        in_specs=[pl.BlockSpec(memory_space=pltpu.VMEM)] * 5,
        out_specs=pl.BlockSpec(memory_space=pltpu.VMEM),
        scratch_shapes=[
            pltpu.VMEM((B, HQ * DH, SKV_LOC), jnp.bfloat16),
            pltpu.VMEM((B, SKV_LOC, HQ * DH), jnp.bfloat16),
            pltpu.SemaphoreType.DMA((4,)),
            pltpu.SemaphoreType.DMA((2,)),
        ],
        compiler_params=(None if _NO_COMM
                         else pltpu.CompilerParams(collective_id=0)),
    )(xb, wqb, kb, vb, wob)


# device time: 55673 ns/iter; 1.3273x vs baseline; 1.3273x over previous
import functools
import os

import jax
import jax.numpy as jnp
from jax import lax
from jax.experimental import pallas as pl
from jax.experimental.pallas import tpu as pltpu

N_DEV = 4
_NO_COMM = os.environ.get("KERNEL_NO_COMM") == "1"

B, SQ, SKV_LOC, HQ, DH = 2, 512, 512, 8, 64
HD = HQ * DH
D_MODEL = 768
BLK = 64
C = 4
CH = SKV_LOC // C


def kernel(x, Wq, K_ext, V_ext, Wo):
    xb = x.astype(jnp.bfloat16)
    wqb = Wq.astype(jnp.bfloat16)
    kb = K_ext.astype(jnp.bfloat16).reshape(B, SKV_LOC, HD).transpose(0, 2, 1)
    vb = V_ext.astype(jnp.bfloat16).reshape(B, SKV_LOC, HD)
    wob = Wo.astype(jnp.bfloat16)

    def body(x_ref, wq_ref, k_ref, v_ref, wo_ref, out_ref,
             kb_buf, vb_buf, q_buf, send_sems, recv_sems):
        my = lax.axis_index("i")
        left = (my - 1) % N_DEV
        right = (my + 1) % N_DEV

        if not _NO_COMM:
            barrier_sem = pltpu.get_barrier_semaphore()
            for nbr in (left, right):
                pl.semaphore_signal(
                    barrier_sem, inc=1,
                    device_id=(nbr,), device_id_type=pl.DeviceIdType.MESH,
                )
            pl.semaphore_wait(barrier_sem, 2)

        def chunk_ops(target, c, send_base):
            ops = []
            for t, src, dst in (
                (0, kb_buf.at[:, :, c * CH:(c + 1) * CH],
                    kb_buf.at[:, :, c * CH:(c + 1) * CH]),
                (1, vb_buf.at[:, c * CH:(c + 1) * CH, :],
                    vb_buf.at[:, c * CH:(c + 1) * CH, :]),
            ):
                ops.append(pltpu.make_async_remote_copy(
                    src_ref=src, dst_ref=dst,
                    send_sem=send_sems.at[send_base + 2 * c + t],
                    recv_sem=recv_sems.at[2 * c + t],
                    device_id=(target,),
                    device_id_type=pl.DeviceIdType.MESH,
                ))
            return ops

        if _NO_COMM:
            kb_buf[...] = k_ref[...]
            vb_buf[...] = v_ref[...]
        else:
            @pl.when(my == 0)
            def _():
                kb_buf[...] = k_ref[...]
                vb_buf[...] = v_ref[...]
                for c in range(C):
                    for op in chunk_ops(1, c, 0) + chunk_ops(3, c, 2 * C):
                        op.start()

        for b in range(B):
            q_buf[b] = jnp.dot(
                x_ref[b], wq_ref[...], preferred_element_type=jnp.float32,
            ).astype(jnp.bfloat16)

        for g in range(C):
            if not _NO_COMM:
                @pl.when(my == 1)
                def _(g=g):
                    for op in chunk_ops(left, g, 0):
                        op.wait_recv()
                    for op in chunk_ops(2, g, 0):
                        op.start()

                @pl.when((my == 2) | (my == 3))
                def _(g=g):
                    for op in chunk_ops(left, g, 0):
                        op.wait_recv()

            ncols = (g + 1) * CH
            rowb = (lax.broadcasted_iota(jnp.int32, (CH, ncols), 0)
                    + g * CH) // BLK
            colb = lax.broadcasted_iota(jnp.int32, (CH, ncols), 1) // BLK
            mask = colb <= rowb

            for b in range(B):
                acc = jnp.zeros((CH, D_MODEL), jnp.float32)
                for h in range(HQ):
                    q_gh = q_buf[b, g * CH:(g + 1) * CH, h * DH:(h + 1) * DH]
                    kt_h = kb_buf[b, h * DH:(h + 1) * DH, 0:ncols]
                    s = jnp.dot(q_gh, kt_h,
                                preferred_element_type=jnp.float32) * 0.125
                    s = jnp.where(mask, s, -1e9)
                    m = jnp.max(s, axis=-1, keepdims=True)
                    w = jnp.exp(s - m)
                    w = w / jnp.sum(w, axis=-1, keepdims=True)
                    v_h = vb_buf[b, 0:ncols, h * DH:(h + 1) * DH]
                    ctx = jnp.dot(w.astype(jnp.bfloat16), v_h,
                                  preferred_element_type=jnp.float32)
                    acc = acc + jnp.dot(
                        ctx.astype(jnp.bfloat16),
                        wo_ref[h * DH:(h + 1) * DH, :],
                        preferred_element_type=jnp.float32)
                out_ref[b, g * CH:(g + 1) * CH, :] = acc

        if not _NO_COMM:
            @pl.when(my == 0)
            def _():
                for c in range(C):
                    for op in chunk_ops(1, c, 0) + chunk_ops(3, c, 2 * C):
                        op.wait_send()

            @pl.when(my == 1)
            def _():
                for c in range(C):
                    for op in chunk_ops(2, c, 0):
                        op.wait_send()

            @functools.partial(pl.run_scoped,
                               second_barrier=pltpu.SemaphoreType.REGULAR)
            def _(second_barrier):
                for nbr in (left, right):
                    pl.semaphore_signal(
                        second_barrier, inc=1,
                        device_id=(nbr,), device_id_type=pl.DeviceIdType.MESH,
                    )
                pl.semaphore_wait(second_barrier, 2)

    return pl.pallas_call(
        body,
        out_shape=jax.ShapeDtypeStruct((B, SQ, D_MODEL), jnp.float32),
        in_specs=[pl.BlockSpec(memory_space=pltpu.VMEM)] * 5,
        out_specs=pl.BlockSpec(memory_space=pltpu.VMEM),
        scratch_shapes=[
            pltpu.VMEM((B, HD, SKV_LOC), jnp.bfloat16),
            pltpu.VMEM((B, SKV_LOC, HD), jnp.bfloat16),
            pltpu.VMEM((B, SQ, HD), jnp.bfloat16),
            pltpu.SemaphoreType.DMA((4 * C,)),
            pltpu.SemaphoreType.DMA((2 * C,)),
        ],
        compiler_params=(None if _NO_COMM
                         else pltpu.CompilerParams(collective_id=0)),
    )(xb, wqb, kb, vb, wob)


# device time: 41537 ns/iter; 1.7789x vs baseline; 1.3403x over previous
import functools
import os

import jax
import jax.numpy as jnp
from jax import lax
from jax.experimental import pallas as pl
from jax.experimental.pallas import tpu as pltpu

N_DEV = 4
_NO_COMM = os.environ.get("KERNEL_NO_COMM") == "1"
_COMM_ONLY = os.environ.get("KERNEL_COMM_ONLY") == "1"

B, SQ, SKV_LOC, HQ, DH = 2, 512, 512, 8, 64
HD = HQ * DH
D_MODEL = 768
BLK = 64
CK = CV = int(os.environ.get("KERNEL_CC", "8"))
CKH = CVH = SKV_LOC // CK
GR = SQ // 2


def kernel(x, Wq, K_ext, V_ext, Wo):
    xb = x.astype(jnp.bfloat16)
    wqb = Wq.astype(jnp.bfloat16)
    kb = K_ext.astype(jnp.bfloat16).reshape(B, SKV_LOC, HD)
    vb = V_ext.astype(jnp.bfloat16).reshape(B, SKV_LOC, HD)
    wob = Wo.astype(jnp.bfloat16)

    def body(x_ref, wq_ref, k_ref, v_ref, wo_ref, out_ref,
             kb_buf, vb_buf, q_buf, part_ctx, send_sems, recv_sems):
        my = lax.axis_index("i")
        left = (my - 1) % N_DEV
        right = (my + 1) % N_DEV

        if not _NO_COMM:
            barrier_sem = pltpu.get_barrier_semaphore()
            for nbr in (left, right):
                pl.semaphore_signal(
                    barrier_sem, inc=1,
                    device_id=(nbr,), device_id_type=pl.DeviceIdType.MESH,
                )
            pl.semaphore_wait(barrier_sem, 2)

        NSEM = CK + CV

        def piece_op(target, kind, i, send_base, from_input=False):
            if kind == "k":
                buf, inp, slot = kb_buf, k_ref, i
            else:
                buf, inp, slot = vb_buf, v_ref, CK + i
            src = (inp if from_input else buf).at[:, i * CKH:(i + 1) * CKH, :]
            dst = buf.at[:, i * CKH:(i + 1) * CKH, :]
            return pltpu.make_async_remote_copy(
                src_ref=src, dst_ref=dst,
                send_sem=send_sems.at[send_base + slot],
                recv_sem=recv_sems.at[slot],
                device_id=(target,),
                device_id_type=pl.DeviceIdType.MESH,
            )

        def half_order(h):
            order = []
            for i in range(h * CK // 2, (h + 1) * CK // 2):
                order += [("k", i), ("v", i)]
            return order

        if _NO_COMM:
            kb_buf[...] = k_ref[...]
            vb_buf[...] = v_ref[...]
        else:
            @pl.when(my == 0)
            def _():
                for h in range(2):
                    for kind, i in half_order(h):
                        piece_op(1, kind, i, 0, from_input=True).start()
                        piece_op(3, kind, i, NSEM, from_input=True).start()
                kb_buf[...] = k_ref[...]
                vb_buf[...] = v_ref[...]

        def comm_wait(h):
            if _NO_COMM:
                return

            @pl.when(my == 1)
            def _():
                for kind, i in half_order(h):
                    piece_op(left, kind, i, 0).wait_recv()
                    piece_op(2, kind, i, 0).start()

            @pl.when((my == 2) | (my == 3))
            def _():
                for kind, i in half_order(h):
                    piece_op(left, kind, i, 0).wait_recv()

        if not _COMM_ONLY:
            for b in range(B):
                q_buf[b] = (jnp.dot(
                    x_ref[b], wq_ref[...], preferred_element_type=jnp.float32,
                ) * 0.125).astype(jnp.bfloat16)

        comm_wait(0)

        mask = (lax.broadcasted_iota(jnp.int32, (GR, GR), 1) // BLK
                <= lax.broadcasted_iota(jnp.int32, (GR, GR), 0) // BLK)

        def attend(b, h, r0, c0, ncols, masked):
            q_gh = q_buf[b, r0:r0 + GR, h * DH:(h + 1) * DH]
            k_h = kb_buf[b, c0:c0 + ncols, h * DH:(h + 1) * DH]
            s = lax.dot_general(q_gh, k_h, (((1,), (1,)), ((), ())),
                                preferred_element_type=jnp.float32)
            e = jnp.exp(s)
            if masked:
                e = jnp.where(mask, e, 0.0)
            l = jnp.sum(e, axis=-1, keepdims=True)
            v_h = vb_buf[b, c0:c0 + ncols, h * DH:(h + 1) * DH]
            ctx = jnp.dot(e.astype(jnp.bfloat16), v_h,
                          preferred_element_type=jnp.float32)
            return ctx, l

        g0_ctx = {b: [] for b in range(B)}
        part_l = []

        def g0_unit(b, h):
            ctx, l = attend(b, h, 0, 0, GR, masked=True)
            g0_ctx[b].append((ctx / l).astype(jnp.bfloat16))

        def g0_store(b):
            out_ref[b, 0:GR, :] = jnp.dot(
                jnp.concatenate(g0_ctx[b], axis=1), wo_ref[...],
                preferred_element_type=jnp.float32)

        def g1a_unit(b, h):
            ctx, l = attend(b, h, GR, 0, GR, masked=False)
            part_ctx[b, :, h * DH:(h + 1) * DH] = ctx
            part_l.append(l)

        thunks = []
        for b in range(B):
            for h in range(HQ):
                thunks.append(functools.partial(g0_unit, b, h))
            thunks.append(functools.partial(g0_store, b))
        for b in range(B):
            for h in range(HQ):
                thunks.append(functools.partial(g1a_unit, b, h))

        pieces1 = half_order(1)
        sched = {max(1, (p + 1) * (len(thunks) - 2) // len(pieces1)): p
                 for p in range(len(pieces1) - 1)}

        for j, thunk in enumerate(thunks):
            if not _NO_COMM and j in sched:
                kind, i = pieces1[sched[j]]

                @pl.when(my == 1)
                def _(kind=kind, i=i):
                    piece_op(left, kind, i, 0).wait_recv()
                    piece_op(2, kind, i, 0).start()
            if not _COMM_ONLY:
                thunk()

        if not _NO_COMM:
            @pl.when(my == 1)
            def _():
                for kind, i in pieces1[len(sched):]:
                    piece_op(left, kind, i, 0).wait_recv()
                    piece_op(2, kind, i, 0).start()

            @pl.when((my == 2) | (my == 3))
            def _():
                for kind, i in pieces1:
                    piece_op(left, kind, i, 0).wait_recv()

        if _COMM_ONLY:
            for b in range(B):
                out_ref[b] = jnp.zeros((SQ, D_MODEL), jnp.float32)
        else:
            for b in range(B):
                ctxs = []
                for h in range(HQ):
                    ctx2, l2 = attend(b, h, GR, GR, GR, masked=True)
                    ctx1 = part_ctx[b, :, h * DH:(h + 1) * DH]
                    l1 = part_l[b * HQ + h]
                    ctxs.append(
                        ((ctx1 + ctx2) / (l1 + l2)).astype(jnp.bfloat16))
                out_ref[b, GR:SQ, :] = jnp.dot(
                    jnp.concatenate(ctxs, axis=1), wo_ref[...],
                    preferred_element_type=jnp.float32)

        if not _NO_COMM:
            @pl.when(my == 0)
            def _():
                for h in range(2):
                    for kind, i in half_order(h):
                        piece_op(1, kind, i, 0, from_input=True).wait_send()
                        piece_op(3, kind, i, NSEM, from_input=True).wait_send()

            @pl.when(my == 1)
            def _():
                for h in range(2):
                    for kind, i in half_order(h):
                        piece_op(2, kind, i, 0).wait_send()

            @functools.partial(pl.run_scoped,
                               second_barrier=pltpu.SemaphoreType.REGULAR)
            def _(second_barrier):
                for nbr in (left, right):
                    pl.semaphore_signal(
                        second_barrier, inc=1,
                        device_id=(nbr,), device_id_type=pl.DeviceIdType.MESH,
                    )
                pl.semaphore_wait(second_barrier, 2)

    return pl.pallas_call(
        body,
        out_shape=jax.ShapeDtypeStruct((B, SQ, D_MODEL), jnp.float32),
        in_specs=[pl.BlockSpec(memory_space=pltpu.VMEM)] * 5,
        out_specs=pl.BlockSpec(memory_space=pltpu.VMEM),
        scratch_shapes=[
            pltpu.VMEM((B, SKV_LOC, HD), jnp.bfloat16),
            pltpu.VMEM((B, SKV_LOC, HD), jnp.bfloat16),
            pltpu.VMEM((B, SQ, HD), jnp.bfloat16),
            pltpu.VMEM((B, GR, HD), jnp.float32),
            pltpu.SemaphoreType.DMA((2 * (CK + CV),)),
            pltpu.SemaphoreType.DMA((CK + CV,)),
        ],
        compiler_params=(None if _NO_COMM
                         else pltpu.CompilerParams(collective_id=0)),
    )(xb, wqb, kb, vb, wob)
